# baseline (speedup 1.0000x reference)
import functools

import jax
import jax.numpy as jnp
from jax import lax
from jax.experimental import pallas as pl
from jax.experimental.pallas import tpu as pltpu
from jax.experimental.pallas import tpu_sc as plsc

_BLOCK_S = 512


def _add_pe_tc_kernel(x_ref, pe_ref, o_ref):
    o_ref[...] = x_ref[...] + pe_ref[...][None, :, :]


def _tc_kernel(inputs, pos_table):
    B, S, D = inputs.shape
    grid = (S // _BLOCK_S,)
    return pl.pallas_call(
        _add_pe_tc_kernel,
        grid=grid,
        in_specs=[
            pl.BlockSpec((B, _BLOCK_S, D), lambda i: (0, i, 0)),
            pl.BlockSpec((_BLOCK_S, D), lambda i: (i, 0)),
        ],
        out_specs=pl.BlockSpec((B, _BLOCK_S, D), lambda i: (0, i, 0)),
        out_shape=jax.ShapeDtypeStruct((B, S, D), inputs.dtype),
        compiler_params=pltpu.CompilerParams(
            dimension_semantics=("parallel",),
        ),
    )(inputs, pos_table)


_CHUNK_W = 16384
_NBUF = 2
_UNROLL = 8


def _make_sc_kernel(NWORDS, TWORDS):
    info = plsc.get_sparse_core_info()
    NC, NS, L = info.num_cores, info.num_subcores, info.num_lanes
    NW = NC * NS
    words_per_w = NWORDS // NW
    n_chunks = words_per_w // _CHUNK_W
    mesh = plsc.VectorSubcoreMesh(core_axis_name="c", subcore_axis_name="s")

    @functools.partial(
        pl.kernel,
        mesh=mesh,
        out_type=jax.ShapeDtypeStruct((NWORDS,), jnp.float32),
        scratch_types=(
            [pltpu.VMEM((_CHUNK_W,), jnp.float32) for _ in range(3 * _NBUF)]
            + [pltpu.SemaphoreType.DMA for _ in range(3 * _NBUF)]
        ),
    )
    def k(x_hbm, t_hbm, o_hbm, *scratch):
        bufx = scratch[:_NBUF]
        buft = scratch[_NBUF:2 * _NBUF]
        bufo = scratch[2 * _NBUF:3 * _NBUF]
        semx = scratch[3 * _NBUF:4 * _NBUF]
        semt = scratch[4 * _NBUF:5 * _NBUF]
        semo = scratch[5 * _NBUF:6 * _NBUF]
        wid = lax.axis_index("s") * NC + lax.axis_index("c")
        base = wid * words_per_w
        tbase = lax.rem(base, TWORDS)

        fills = [None] * n_chunks
        stores = [None] * n_chunks

        def start_fill(c):
            b = c % _NBUF
            fx = pltpu.async_copy(
                x_hbm.at[pl.ds(base + c * _CHUNK_W, _CHUNK_W)], bufx[b], semx[b])
            ft = pltpu.async_copy(
                t_hbm.at[pl.ds(tbase + c * _CHUNK_W, _CHUNK_W)], buft[b], semt[b])
            fills[c] = (fx, ft)

        def run_add(b):
            @plsc.parallel_loop(0, _CHUNK_W, step=L, unroll=_UNROLL)
            def _(i):
                s = pl.ds(i, L)
                bufo[b][s] = bufx[b][s] + buft[b][s]

        for c in range(min(_NBUF, n_chunks)):
            start_fill(c)
        for c in range(n_chunks):
            b = c % _NBUF
            fills[c][0].wait()
            fills[c][1].wait()
            if c - _NBUF >= 0:
                stores[c - _NBUF].wait()
            run_add(b)
            stores[c] = pltpu.async_copy(
                bufo[b], o_hbm.at[pl.ds(base + c * _CHUNK_W, _CHUNK_W)], semo[b])
            if c + _NBUF < n_chunks:
                start_fill(c + _NBUF)
        for c in range(max(0, n_chunks - _NBUF), n_chunks):
            stores[c].wait()

    return k


def kernel(inputs, pos_table):
    B, S, D = inputs.shape
    x = inputs.reshape(B * S * D)
    t = pos_table.reshape(S * D)
    out = _make_sc_kernel(B * S * D, S * D)(x, t)
    return out.reshape(B, S, D)


# speedup vs baseline: 4.5882x; 4.5882x over previous
import jax
import jax.numpy as jnp
from jax.experimental import pallas as pl
from jax.experimental.pallas import tpu as pltpu

_BLOCK_S = 512


def _add_pe_kernel(x_ref, pe_ref, o_ref):
    o_ref[...] = x_ref[...] + pe_ref[...][None, :, :]


def kernel(inputs, pos_table):
    B, S, D = inputs.shape
    grid = (S // _BLOCK_S,)
    return pl.pallas_call(
        _add_pe_kernel,
        grid=grid,
        in_specs=[
            pl.BlockSpec((B, _BLOCK_S, D), lambda i: (0, i, 0)),
            pl.BlockSpec((_BLOCK_S, D), lambda i: (i, 0)),
        ],
        out_specs=pl.BlockSpec((B, _BLOCK_S, D), lambda i: (0, i, 0)),
        out_shape=jax.ShapeDtypeStruct((B, S, D), inputs.dtype),
        compiler_params=pltpu.CompilerParams(
            dimension_semantics=("parallel",),
        ),
    )(inputs, pos_table)


# speedup vs baseline: 4.6279x; 1.0087x over previous
import jax
import jax.numpy as jnp
from jax.experimental import pallas as pl
from jax.experimental.pallas import tpu as pltpu

_BLOCK_S = 2048


def _add_pe_kernel(x_ref, pe_ref, o_ref):
    o_ref[...] = x_ref[...] + pe_ref[...]


def kernel(inputs, pos_table):
    B, S, D = inputs.shape
    grid = (S // _BLOCK_S, B)
    return pl.pallas_call(
        _add_pe_kernel,
        grid=grid,
        in_specs=[
            pl.BlockSpec((1, _BLOCK_S, D), lambda i, j: (j, i, 0)),
            pl.BlockSpec((_BLOCK_S, D), lambda i, j: (i, 0)),
        ],
        out_specs=pl.BlockSpec((1, _BLOCK_S, D), lambda i, j: (j, i, 0)),
        out_shape=jax.ShapeDtypeStruct((B, S, D), inputs.dtype),
        compiler_params=pltpu.CompilerParams(
            dimension_semantics=("arbitrary", "arbitrary"),
        ),
    )(inputs, pos_table)
